# Initial kernel scaffold; baseline (speedup 1.0000x reference)
import jax
import jax.numpy as jnp
from jax.experimental import pallas as pl


def kernel(x, edge_index, W1, att_src1, att_dst1, b1, bn_gamma, bn_beta, W2, att_src2, att_dst2, b2):
    raise NotImplementedError("write your pallas kernel here")


# speedup vs baseline: 1.0000x; 1.0000x over previous
import jax
import jax.numpy as jnp
from jax.experimental import pallas as pl

N = 10000
HEADS = 8
C1 = 8
D_OUT = 16
NEG_SLOPE = 0.2


def _gat_layer(x, src, dst, W, att_src, att_dst, b, heads, ch, concat):
    n = x.shape[0]
    h = (x @ W).reshape(n, heads, ch)
    a_src = (h * att_src[None, :, :]).sum(-1)
    a_dst = (h * att_dst[None, :, :]).sum(-1)
    alpha = a_src[src] + a_dst[dst]
    alpha = jax.nn.leaky_relu(alpha, NEG_SLOPE)
    amax = jax.ops.segment_max(alpha, dst, num_segments=n)
    amax = jnp.where(jnp.isfinite(amax), amax, 0.0)
    ex = jnp.exp(alpha - amax[dst])
    denom = jax.ops.segment_sum(ex, dst, num_segments=n)
    att = ex / (denom[dst] + 1e-16)
    msg = h[src] * att[:, :, None]
    out = jax.ops.segment_sum(msg, dst, num_segments=n)
    if concat:
        out = out.reshape(n, heads * ch)
    else:
        out = out.mean(axis=1)
    return out + b


def kernel(x, edge_index, W1, att_src1, att_dst1, b1, bn_gamma, bn_beta, W2, att_src2, att_dst2, b2):
    loops = jnp.arange(x.shape[0], dtype=edge_index.dtype)
    src = jnp.concatenate([edge_index[0], loops])
    dst = jnp.concatenate([edge_index[1], loops])
    h = _gat_layer(x, src, dst, W1, att_src1, att_dst1, b1, HEADS, C1, True)
    h = jax.nn.relu(h)
    mean = h.mean(axis=0)
    var = h.var(axis=0)
    h = (h - mean) / jnp.sqrt(var + 1e-5) * bn_gamma + bn_beta
    out = _gat_layer(h, src, dst, W2, att_src2, att_dst2, b2, 1, D_OUT, False)
    return out


# speedup vs baseline: 58.2354x; 58.2349x over previous
import jax
import jax.numpy as jnp
from jax import lax
from jax.experimental import pallas as pl
from jax.experimental.pallas import tpu as pltpu
from jax.experimental.pallas import tpu_sc as plsc

N = 10000
E = 320000
D_IN = 128
HEADS = 8
C1 = 8
D_OUT = 16
NEG_SLOPE = 0.2

NC = 2
NS = 16
NW = NC * NS
CH = 128
NP = 10240
ROWS_PER_TILE = NP // NS
EP = 331776
PER_W = EP // NW
CHUNKS = PER_W // CH
W1ACC = 80
W2ACC = 32


def _leaky(t):
    return jnp.where(t >= 0, t, t * NEG_SLOPE)


def _tc1_body(x_ref, w1_ref, as_ref, ad_ref, h1_ref, tabs_ref, tabd_ref, g_ref):
    x = x_ref[...]
    h1 = jnp.dot(x, w1_ref[...], preferred_element_type=jnp.float32)
    tabs = jnp.dot(h1, as_ref[...], preferred_element_type=jnp.float32)
    tabd = jnp.dot(h1, ad_ref[...], preferred_element_type=jnp.float32)
    h1_ref[...] = h1
    tabs_ref[...] = tabs
    tabd_ref[...] = tabd
    g = _leaky(jnp.max(tabs, axis=0, keepdims=True)
               + jnp.max(tabd, axis=0, keepdims=True))
    g_ref[...] = jnp.broadcast_to(g, (8, 16))


def _tc1(xp, W1, AsD, AdD):
    return pl.pallas_call(
        _tc1_body,
        out_shape=[
            jax.ShapeDtypeStruct((NP, HEADS * C1), jnp.float32),
            jax.ShapeDtypeStruct((NP, 16), jnp.float32),
            jax.ShapeDtypeStruct((NP, 16), jnp.float32),
            jax.ShapeDtypeStruct((8, 16), jnp.float32),
        ],
    )(xp, W1, AsD, AdD)


def _sc1_body(src_hbm, dst_hbm, tabs_hbm, tabd_hbm, h1_hbm, g_hbm, out_hbm,
              srcbuf, dstbuf, srows, drows, hrows, exbuf, msgbuf, gbuf, acc,
              sem1, sem2, sem3):
    c = lax.axis_index("c")
    s = lax.axis_index("s")
    wid = s * NC + c
    iota = lax.iota(jnp.int32, 16)
    zero16 = jnp.zeros((16,), jnp.float32)

    pltpu.sync_copy(g_hbm, gbuf)
    gvec = gbuf[...]

    def _zb(t, _):
        msgbuf[t // 5, pl.ds((t % 5) * 16, 16)] = zero16
        return 0
    lax.fori_loop(0, CH * 5, _zb, 0)
    for q in range(ROWS_PER_TILE // CH):
        pltpu.sync_copy(msgbuf, acc.at[pl.ds(s * ROWS_PER_TILE + q * CH, CH)])
    plsc.subcore_barrier()

    hvecs = [2 * q + (iota >> 3) for q in range(4)]

    def _chunk(t, _):
        off = pl.multiple_of(wid * PER_W + t * CH, CH)
        pltpu.sync_copy(src_hbm.at[pl.ds(off, CH)], srcbuf)
        pltpu.sync_copy(dst_hbm.at[pl.ds(off, CH)], dstbuf)
        cp1 = pltpu.async_copy(tabs_hbm.at[srcbuf], srows, sem1)
        cp2 = pltpu.async_copy(tabd_hbm.at[dstbuf], drows, sem2)
        cp3 = pltpu.async_copy(h1_hbm.at[srcbuf], hrows, sem3)
        cp1.wait()
        cp2.wait()

        def _exb(k, _):
            rs = srows[k, pl.ds(0, 16)]
            rd = drows[k, pl.ds(0, 16)]
            e = jnp.exp(_leaky(rs + rd) - gvec)
            msgbuf[k, pl.ds(64, 16)] = e
            exbuf[pl.ds(16 * k, 16)] = e
            return 0
        lax.fori_loop(0, CH, _exb, 0)
        cp3.wait()

        def _msg(k, _):
            for q in range(4):
                eb = plsc.load_gather(exbuf, [16 * k + hvecs[q]])
                hv = hrows[k, pl.ds(q * 16, 16)]
                msgbuf[k, pl.ds(q * 16, 16)] = eb * hv
            return 0
        lax.fori_loop(0, CH, _msg, 0)

        pltpu.sync_copy(msgbuf, acc.at[dstbuf], add=True)
        return 0

    lax.fori_loop(0, CHUNKS, _chunk, 0)
    plsc.subcore_barrier()
    pltpu.sync_copy(acc.at[pl.ds(s * ROWS_PER_TILE, ROWS_PER_TILE)],
                    out_hbm.at[c, pl.ds(s * ROWS_PER_TILE, ROWS_PER_TILE)])


def _sc1(src, dst, tabs, tabd, h1, g16):
    mesh = plsc.VectorSubcoreMesh(core_axis_name="c", subcore_axis_name="s")
    f = pl.kernel(
        _sc1_body,
        out_type=jax.ShapeDtypeStruct((NC, NP, W1ACC), jnp.float32),
        mesh=mesh,
        compiler_params=pltpu.CompilerParams(needs_layout_passes=False,
                                             use_tc_tiling_on_sc=False),
        scratch_types=[
            pltpu.VMEM((CH,), jnp.int32),
            pltpu.VMEM((CH,), jnp.int32),
            pltpu.VMEM((CH, 16), jnp.float32),
            pltpu.VMEM((CH, 16), jnp.float32),
            pltpu.VMEM((CH, HEADS * C1), jnp.float32),
            pltpu.VMEM((CH * 16,), jnp.float32),
            pltpu.VMEM((CH, W1ACC), jnp.float32),
            pltpu.VMEM((16,), jnp.float32),
            pltpu.VMEM_SHARED((NP, W1ACC), jnp.float32),
            pltpu.SemaphoreType.DMA,
            pltpu.SemaphoreType.DMA,
            pltpu.SemaphoreType.DMA,
        ],
    )
    return f(src, dst, tabs, tabd, h1, g16)


def _tc2_body(p_ref, b1_ref, gam_ref, bet_ref, w2_ref, a2w_ref, e8_ref,
              h2_ref, a2_ref, g_ref):
    m = p_ref[0] + p_ref[1]
    num = m[:, :64]
    den = m[:, 64:72]
    r = 1.0 / (den + 1e-16)
    h = num * jnp.dot(r, e8_ref[...], preferred_element_type=jnp.float32)
    h = h + b1_ref[...][None, :]
    h = jnp.maximum(h, 0.0)
    rows = lax.broadcasted_iota(jnp.int32, (NP, 1), 0)
    mask = rows < N
    hm = jnp.where(mask, h, 0.0)
    mean = jnp.sum(hm, axis=0, keepdims=True) / N
    d = jnp.where(mask, h - mean, 0.0)
    var = jnp.sum(d * d, axis=0, keepdims=True) / N
    hbn = (h - mean) * lax.rsqrt(var + 1e-5) * gam_ref[...][None, :] \
        + bet_ref[...][None, :]
    hbn = jnp.where(mask, hbn, 0.0)
    h2 = jnp.dot(hbn, w2_ref[...], preferred_element_type=jnp.float32)
    a2 = jnp.dot(h2, a2w_ref[...], preferred_element_type=jnp.float32)
    h2_ref[...] = h2
    a2_ref[...] = a2
    g = _leaky(jnp.max(a2[:, 0:1], axis=0, keepdims=True)
               + jnp.max(a2[:, 1:2], axis=0, keepdims=True))
    g_ref[...] = jnp.broadcast_to(g, (8, 16))


def _tc2(part1, b1, bn_gamma, bn_beta, W2, A2, E8):
    return pl.pallas_call(
        _tc2_body,
        out_shape=[
            jax.ShapeDtypeStruct((NP, D_OUT), jnp.float32),
            jax.ShapeDtypeStruct((NP, 8), jnp.float32),
            jax.ShapeDtypeStruct((8, 16), jnp.float32),
        ],
    )(part1, b1, bn_gamma, bn_beta, W2, A2, E8)


def _sc2_body(src_hbm, dst_hbm, asrc_hbm, adst_hbm, h2_hbm, g_hbm, out_hbm,
              srcbuf, dstbuf, asrcbuf, adstbuf, hrows, exbuf, msgbuf, gbuf,
              acc, sem1):
    c = lax.axis_index("c")
    s = lax.axis_index("s")
    wid = s * NC + c
    iota = lax.iota(jnp.int32, 16)
    zero16 = jnp.zeros((16,), jnp.float32)

    pltpu.sync_copy(g_hbm, gbuf)
    pltpu.sync_copy(asrc_hbm, asrcbuf)
    pltpu.sync_copy(adst_hbm, adstbuf)
    g0 = gbuf[...][0]

    def _zb(t, _):
        msgbuf[t >> 1, pl.ds((t & 1) * 16, 16)] = zero16
        return 0
    lax.fori_loop(0, CH * 2, _zb, 0)
    for q in range(ROWS_PER_TILE // CH):
        pltpu.sync_copy(msgbuf, acc.at[pl.ds(s * ROWS_PER_TILE + q * CH, CH)])
    plsc.subcore_barrier()

    def _chunk(t, _):
        off = pl.multiple_of(wid * PER_W + t * CH, CH)
        pltpu.sync_copy(src_hbm.at[pl.ds(off, CH)], srcbuf)
        pltpu.sync_copy(dst_hbm.at[pl.ds(off, CH)], dstbuf)
        cp1 = pltpu.async_copy(h2_hbm.at[srcbuf], hrows, sem1)

        def _exb(i, _):
            sv = srcbuf[pl.ds(i * 16, 16)]
            dv = dstbuf[pl.ds(i * 16, 16)]
            u = plsc.load_gather(asrcbuf, [sv])
            v = plsc.load_gather(adstbuf, [dv])
            e = jnp.exp(_leaky(u + v) - g0)
            exbuf[pl.ds(i * 16, 16)] = e
            return 0
        lax.fori_loop(0, CH // 16, _exb, 0)
        cp1.wait()

        def _msg(k, _):
            eb = plsc.load_gather(exbuf, [jnp.full((16,), k, jnp.int32)])
            hv = hrows[k, pl.ds(0, 16)]
            msgbuf[k, pl.ds(0, 16)] = eb * hv
            msgbuf[k, pl.ds(16, 16)] = eb
            return 0
        lax.fori_loop(0, CH, _msg, 0)

        pltpu.sync_copy(msgbuf, acc.at[dstbuf], add=True)
        return 0

    lax.fori_loop(0, CHUNKS, _chunk, 0)
    plsc.subcore_barrier()
    pltpu.sync_copy(acc.at[pl.ds(s * ROWS_PER_TILE, ROWS_PER_TILE)],
                    out_hbm.at[c, pl.ds(s * ROWS_PER_TILE, ROWS_PER_TILE)])


def _sc2(src, dst, asrc2, adst2, h2, g16):
    mesh = plsc.VectorSubcoreMesh(core_axis_name="c", subcore_axis_name="s")
    f = pl.kernel(
        _sc2_body,
        out_type=jax.ShapeDtypeStruct((NC, NP, W2ACC), jnp.float32),
        mesh=mesh,
        compiler_params=pltpu.CompilerParams(needs_layout_passes=False,
                                             use_tc_tiling_on_sc=False),
        scratch_types=[
            pltpu.VMEM((CH,), jnp.int32),
            pltpu.VMEM((CH,), jnp.int32),
            pltpu.VMEM((NP,), jnp.float32),
            pltpu.VMEM((NP,), jnp.float32),
            pltpu.VMEM((CH, D_OUT), jnp.float32),
            pltpu.VMEM((CH,), jnp.float32),
            pltpu.VMEM((CH, W2ACC), jnp.float32),
            pltpu.VMEM((16,), jnp.float32),
            pltpu.VMEM_SHARED((NP, W2ACC), jnp.float32),
            pltpu.SemaphoreType.DMA,
        ],
    )
    return f(src, dst, asrc2, adst2, h2, g16)


def _tc3_body(p_ref, b2_ref, out_ref):
    m = p_ref[0] + p_ref[1]
    num = m[:, :16]
    den = m[:, 16:17]
    out_ref[...] = num / (den + 1e-16) + b2_ref[...][None, :]


def _tc3(part2, b2):
    return pl.pallas_call(
        _tc3_body,
        out_shape=jax.ShapeDtypeStruct((NP, D_OUT), jnp.float32),
    )(part2, b2)


def kernel(x, edge_index, W1, att_src1, att_dst1, b1, bn_gamma, bn_beta,
           W2, att_src2, att_dst2, b2):
    xp = jnp.pad(x, ((0, NP - N), (0, 0)))
    ei = edge_index.astype(jnp.int32)
    loops = jnp.arange(N, dtype=jnp.int32)
    padi = jnp.full((EP - E - N,), N, jnp.int32)
    src = jnp.concatenate([ei[0], loops, padi])
    dst = jnp.concatenate([ei[1], loops, padi])

    eye8 = jnp.eye(8, dtype=jnp.float32)
    As = (att_src1[:, :, None] * eye8[:, None, :]).reshape(64, 8)
    Ad = (att_dst1[:, :, None] * eye8[:, None, :]).reshape(64, 8)
    AsD = jnp.concatenate([As, As], axis=1)
    AdD = jnp.concatenate([Ad, Ad], axis=1)
    E8 = jnp.repeat(eye8, 8, axis=1)
    A2 = jnp.zeros((D_OUT, 8), jnp.float32)
    A2 = A2.at[:, 0].set(att_src2[0]).at[:, 1].set(att_dst2[0])

    h1, tabs, tabd, g88 = _tc1(xp, W1, AsD, AdD)
    g16 = g88[0]
    part1 = _sc1(src, dst, tabs, tabd, h1, g16)
    h2, a2, g2_88 = _tc2(part1, b1, bn_gamma, bn_beta, W2, A2, E8)
    g2_16 = g2_88[0]
    part2 = _sc2(src, dst, a2[:, 0], a2[:, 1], h2, g2_16)
    out = _tc3(part2, b2)
    return out[:N]


# speedup vs baseline: 90.8145x; 1.5594x over previous
import jax
import jax.numpy as jnp
from jax import lax
from jax.experimental import pallas as pl
from jax.experimental.pallas import tpu as pltpu
from jax.experimental.pallas import tpu_sc as plsc

N = 10000
E = 320000
D_IN = 128
HEADS = 8
C1 = 8
D_OUT = 16
NEG_SLOPE = 0.2

NC = 2
NS = 16
NW = NC * NS
CH = 128
NP = 10240
ROWS_PER_TILE = NP // NS
CHUNKS = 82
EP = NW * CHUNKS * CH
PER_W = CHUNKS * CH
W1ACC = 80
W2ACC = 32


def _leaky(t):
    return jnp.where(t >= 0, t, t * NEG_SLOPE)


def _tc1_body(x_ref, w1t_ref, as_ref, ad_ref, h1t_ref, tabs_ref, tabd_ref,
              g_ref):
    x = x_ref[...]
    h1t = jnp.dot(x, w1t_ref[...], preferred_element_type=jnp.float32)
    tabs = jnp.dot(h1t, as_ref[...], preferred_element_type=jnp.float32)
    tabd = jnp.dot(h1t, ad_ref[...], preferred_element_type=jnp.float32)
    h1t_ref[...] = h1t
    tabs_ref[...] = tabs
    tabd_ref[...] = tabd
    g = _leaky(jnp.max(tabs, axis=0, keepdims=True)
               + jnp.max(tabd, axis=0, keepdims=True))
    g_ref[...] = jnp.broadcast_to(g, (8, 16))


def _tc1(xp, W1T, AsT, AdT):
    return pl.pallas_call(
        _tc1_body,
        out_shape=[
            jax.ShapeDtypeStruct((NP, HEADS * C1), jnp.float32),
            jax.ShapeDtypeStruct((NP, 16), jnp.float32),
            jax.ShapeDtypeStruct((NP, 16), jnp.float32),
            jax.ShapeDtypeStruct((8, 16), jnp.float32),
        ],
    )(xp, W1T, AsT, AdT)


def _sc1_body(src_hbm, dst_hbm, tabs_hbm, tabd_hbm, h1t_hbm, g_hbm, out_hbm,
              srcidx, dstidx, srows, drows, hrows, msgbuf, gbuf, acc,
              gsem0, gsem1, ssem0, ssem1):
    c = lax.axis_index("c")
    s = lax.axis_index("s")
    wid = s * NC + c
    zero16 = jnp.zeros((16,), jnp.float32)
    gsems = [gsem0, gsem1]
    ssems = [ssem0, ssem1]

    pltpu.sync_copy(g_hbm, gbuf)
    gvec = gbuf[...]
    pltpu.sync_copy(src_hbm.at[wid], srcidx)
    pltpu.sync_copy(dst_hbm.at[wid], dstidx)

    mb0 = msgbuf.at[0]

    def _zb(t, _):
        mb0[t // 5, pl.ds((t % 5) * 16, 16)] = zero16
        return 0
    lax.fori_loop(0, CH * 5, _zb, 0)
    for q in range(ROWS_PER_TILE // CH):
        pltpu.sync_copy(mb0, acc.at[pl.ds(s * ROWS_PER_TILE + q * CH, CH)])
    plsc.subcore_barrier()

    def _issue(t, b):
        pltpu.async_copy(tabs_hbm.at[srcidx.at[t]], srows.at[b], gsems[b])
        pltpu.async_copy(tabd_hbm.at[dstidx.at[t]], drows.at[b], gsems[b])
        pltpu.async_copy(h1t_hbm.at[srcidx.at[t]], hrows.at[b], gsems[b])

    def _wait(t, b):
        pltpu.make_async_copy(tabs_hbm.at[srcidx.at[t]], srows.at[b],
                              gsems[b]).wait()
        pltpu.make_async_copy(tabd_hbm.at[dstidx.at[t]], drows.at[b],
                              gsems[b]).wait()
        pltpu.make_async_copy(h1t_hbm.at[srcidx.at[t]], hrows.at[b],
                              gsems[b]).wait()

    _issue(0, 0)
    _issue(1, 1)

    def _step(t2, _):
        for b in range(2):
            t = 2 * t2 + b
            sr = srows.at[b]
            dr = drows.at[b]
            hr = hrows.at[b]
            mb = msgbuf.at[b]
            _wait(t, b)

            @pl.when(t >= 2)
            def _():
                pltpu.make_async_copy(mb, acc.at[dstidx.at[t]],
                                      ssems[b]).wait()

            def _edge(k, _):
                rs = sr[k, pl.ds(0, 16)]
                rd = dr[k, pl.ds(0, 16)]
                e = jnp.exp(_leaky(rs + rd) - gvec)
                mb[k, pl.ds(64, 16)] = e
                for q in range(4):
                    hv = hr[k, pl.ds(q * 16, 16)]
                    mb[k, pl.ds(q * 16, 16)] = e * hv
                return 0
            lax.fori_loop(0, CH, _edge, 0)

            pltpu.async_copy(mb, acc.at[dstidx.at[t]], ssems[b], add=True)

            @pl.when(t + 2 < CHUNKS)
            def _():
                _issue(t + 2, b)
        return 0

    lax.fori_loop(0, CHUNKS // 2, _step, 0)
    for b in range(2):
        pltpu.make_async_copy(msgbuf.at[b], acc.at[dstidx.at[CHUNKS - 2 + b]],
                              ssems[b]).wait()
    plsc.subcore_barrier()
    pltpu.sync_copy(acc.at[pl.ds(s * ROWS_PER_TILE, ROWS_PER_TILE)],
                    out_hbm.at[c, pl.ds(s * ROWS_PER_TILE, ROWS_PER_TILE)])


def _sc1(src, dst, tabs, tabd, h1t, g16):
    mesh = plsc.VectorSubcoreMesh(core_axis_name="c", subcore_axis_name="s")
    f = pl.kernel(
        _sc1_body,
        out_type=jax.ShapeDtypeStruct((NC, NP, W1ACC), jnp.float32),
        mesh=mesh,
        compiler_params=pltpu.CompilerParams(needs_layout_passes=False,
                                             use_tc_tiling_on_sc=False),
        scratch_types=[
            pltpu.VMEM((CHUNKS, CH), jnp.int32),
            pltpu.VMEM((CHUNKS, CH), jnp.int32),
            pltpu.VMEM((2, CH, 16), jnp.float32),
            pltpu.VMEM((2, CH, 16), jnp.float32),
            pltpu.VMEM((2, CH, HEADS * C1), jnp.float32),
            pltpu.VMEM((2, CH, W1ACC), jnp.float32),
            pltpu.VMEM((16,), jnp.float32),
            pltpu.VMEM_SHARED((NP, W1ACC), jnp.float32),
            pltpu.SemaphoreType.DMA,
            pltpu.SemaphoreType.DMA,
            pltpu.SemaphoreType.DMA,
            pltpu.SemaphoreType.DMA,
        ],
    )
    return f(src, dst, tabs, tabd, h1t, g16)


def _tc2_body(p_ref, b1_ref, gam_ref, bet_ref, w2_ref, a2w_ref, e8t_ref,
              perm_ref, h2_ref, a2_ref, g_ref):
    m = p_ref[0] + p_ref[1]
    num = m[:, :64]
    den = m[:, 64:72]
    r = 1.0 / (den + 1e-16)
    hcm = num * jnp.dot(r, e8t_ref[...], preferred_element_type=jnp.float32)
    h = jnp.dot(hcm, perm_ref[...], preferred_element_type=jnp.float32)
    h = h + b1_ref[...][None, :]
    h = jnp.maximum(h, 0.0)
    rows = lax.broadcasted_iota(jnp.int32, (NP, 1), 0)
    mask = rows < N
    hm = jnp.where(mask, h, 0.0)
    mean = jnp.sum(hm, axis=0, keepdims=True) / N
    d = jnp.where(mask, h - mean, 0.0)
    var = jnp.sum(d * d, axis=0, keepdims=True) / N
    hbn = (h - mean) * lax.rsqrt(var + 1e-5) * gam_ref[...][None, :] \
        + bet_ref[...][None, :]
    hbn = jnp.where(mask, hbn, 0.0)
    h2 = jnp.dot(hbn, w2_ref[...], preferred_element_type=jnp.float32)
    a2 = jnp.dot(h2, a2w_ref[...], preferred_element_type=jnp.float32)
    h2_ref[...] = h2
    a2_ref[...] = a2
    g = _leaky(jnp.max(a2[:, 0:1], axis=0, keepdims=True)
               + jnp.max(a2[:, 1:2], axis=0, keepdims=True))
    g_ref[...] = jnp.broadcast_to(g, (8, 16))


def _tc2(part1, b1, bn_gamma, bn_beta, W2, A2, E8T, P):
    return pl.pallas_call(
        _tc2_body,
        out_shape=[
            jax.ShapeDtypeStruct((NP, D_OUT), jnp.float32),
            jax.ShapeDtypeStruct((NP, 8), jnp.float32),
            jax.ShapeDtypeStruct((8, 16), jnp.float32),
        ],
    )(part1, b1, bn_gamma, bn_beta, W2, A2, E8T, P)


def _sc2_body(src_hbm, dst_hbm, asrc_hbm, adst_hbm, h2_hbm, g_hbm, out_hbm,
              srcidx, dstidx, asrcbuf, adstbuf, hrows, exbuf, msgbuf, gbuf,
              acc, gsem0, gsem1, ssem0, ssem1):
    c = lax.axis_index("c")
    s = lax.axis_index("s")
    wid = s * NC + c
    zero16 = jnp.zeros((16,), jnp.float32)
    gsems = [gsem0, gsem1]
    ssems = [ssem0, ssem1]

    pltpu.sync_copy(g_hbm, gbuf)
    pltpu.sync_copy(asrc_hbm, asrcbuf)
    pltpu.sync_copy(adst_hbm, adstbuf)
    g0 = gbuf[...][0]
    pltpu.sync_copy(src_hbm.at[wid], srcidx)
    pltpu.sync_copy(dst_hbm.at[wid], dstidx)

    mb0 = msgbuf.at[0]

    def _zb(t, _):
        mb0[t >> 1, pl.ds((t & 1) * 16, 16)] = zero16
        return 0
    lax.fori_loop(0, CH * 2, _zb, 0)
    for q in range(ROWS_PER_TILE // CH):
        pltpu.sync_copy(mb0, acc.at[pl.ds(s * ROWS_PER_TILE + q * CH, CH)])
    plsc.subcore_barrier()

    def _issue(t, b):
        pltpu.async_copy(h2_hbm.at[srcidx.at[t]], hrows.at[b], gsems[b])

    def _wait(t, b):
        pltpu.make_async_copy(h2_hbm.at[srcidx.at[t]], hrows.at[b],
                              gsems[b]).wait()

    _issue(0, 0)
    _issue(1, 1)

    def _step(t2, _):
        for b in range(2):
            t = 2 * t2 + b
            hr = hrows.at[b]
            mb = msgbuf.at[b]

            def _exb(i, _):
                sv = srcidx[t, pl.ds(i * 16, 16)]
                dv = dstidx[t, pl.ds(i * 16, 16)]
                u = plsc.load_gather(asrcbuf, [sv])
                v = plsc.load_gather(adstbuf, [dv])
                e = jnp.exp(_leaky(u + v) - g0)
                exbuf[pl.ds(b * CH + i * 16, 16)] = e
                return 0
            lax.fori_loop(0, CH // 16, _exb, 0)
            _wait(t, b)

            @pl.when(t >= 2)
            def _():
                pltpu.make_async_copy(mb, acc.at[dstidx.at[t]],
                                      ssems[b]).wait()

            def _msg(k, _):
                eb = plsc.load_gather(
                    exbuf, [jnp.full((16,), b * CH + k, jnp.int32)])
                hv = hr[k, pl.ds(0, 16)]
                mb[k, pl.ds(0, 16)] = eb * hv
                mb[k, pl.ds(16, 16)] = eb
                return 0
            lax.fori_loop(0, CH, _msg, 0)

            pltpu.async_copy(mb, acc.at[dstidx.at[t]], ssems[b], add=True)

            @pl.when(t + 2 < CHUNKS)
            def _():
                _issue(t + 2, b)
        return 0

    lax.fori_loop(0, CHUNKS // 2, _step, 0)
    for b in range(2):
        pltpu.make_async_copy(msgbuf.at[b], acc.at[dstidx.at[CHUNKS - 2 + b]],
                              ssems[b]).wait()
    plsc.subcore_barrier()
    pltpu.sync_copy(acc.at[pl.ds(s * ROWS_PER_TILE, ROWS_PER_TILE)],
                    out_hbm.at[c, pl.ds(s * ROWS_PER_TILE, ROWS_PER_TILE)])


def _sc2(src, dst, asrc2, adst2, h2, g16):
    mesh = plsc.VectorSubcoreMesh(core_axis_name="c", subcore_axis_name="s")
    f = pl.kernel(
        _sc2_body,
        out_type=jax.ShapeDtypeStruct((NC, NP, W2ACC), jnp.float32),
        mesh=mesh,
        compiler_params=pltpu.CompilerParams(needs_layout_passes=False,
                                             use_tc_tiling_on_sc=False),
        scratch_types=[
            pltpu.VMEM((CHUNKS, CH), jnp.int32),
            pltpu.VMEM((CHUNKS, CH), jnp.int32),
            pltpu.VMEM((NP,), jnp.float32),
            pltpu.VMEM((NP,), jnp.float32),
            pltpu.VMEM((2, CH, D_OUT), jnp.float32),
            pltpu.VMEM((2 * CH,), jnp.float32),
            pltpu.VMEM((2, CH, W2ACC), jnp.float32),
            pltpu.VMEM((16,), jnp.float32),
            pltpu.VMEM_SHARED((NP, W2ACC), jnp.float32),
            pltpu.SemaphoreType.DMA,
            pltpu.SemaphoreType.DMA,
            pltpu.SemaphoreType.DMA,
            pltpu.SemaphoreType.DMA,
        ],
    )
    return f(src, dst, asrc2, adst2, h2, g16)


def _tc3_body(p_ref, b2_ref, out_ref):
    m = p_ref[0] + p_ref[1]
    num = m[:, :16]
    den = m[:, 16:17]
    out_ref[...] = num / (den + 1e-16) + b2_ref[...][None, :]


def _tc3(part2, b2):
    return pl.pallas_call(
        _tc3_body,
        out_shape=jax.ShapeDtypeStruct((NP, D_OUT), jnp.float32),
    )(part2, b2)


def kernel(x, edge_index, W1, att_src1, att_dst1, b1, bn_gamma, bn_beta,
           W2, att_src2, att_dst2, b2):
    xp = jnp.pad(x, ((0, NP - N), (0, 0)))
    ei = edge_index.astype(jnp.int32)
    loops = jnp.arange(N, dtype=jnp.int32)
    padi = jnp.full((EP - E - N,), N, jnp.int32)
    src = jnp.concatenate([ei[0], loops, padi]).reshape(NW, CHUNKS, CH)
    dst = jnp.concatenate([ei[1], loops, padi]).reshape(NW, CHUNKS, CH)

    eye8 = jnp.eye(8, dtype=jnp.float32)
    P = (eye8[:, None, None, :] * eye8[None, :, :, None]).reshape(64, 64)
    W1T = jnp.dot(W1, P)
    AsT = (att_src1.T[:, :, None] * eye8[None, :, :]).reshape(64, 8)
    AdT = (att_dst1.T[:, :, None] * eye8[None, :, :]).reshape(64, 8)
    AsTD = jnp.concatenate([AsT, AsT], axis=1)
    AdTD = jnp.concatenate([AdT, AdT], axis=1)
    E8T = jnp.tile(eye8, (1, 8))
    A2 = jnp.zeros((D_OUT, 8), jnp.float32)
    A2 = A2.at[:, 0].set(att_src2[0]).at[:, 1].set(att_dst2[0])

    h1t, tabs, tabd, g88 = _tc1(xp, W1T, AsTD, AdTD)
    g16 = g88[0]
    part1 = _sc1(src, dst, tabs, tabd, h1t, g16)
    h2, a2, g2_88 = _tc2(part1, b1, bn_gamma, bn_beta, W2, A2, E8T, P)
    g2_16 = g2_88[0]
    part2 = _sc2(src, dst, a2[:, 0], a2[:, 1], h2, g2_16)
    out = _tc3(part2, b2)
    return out[:N]


# speedup vs baseline: 161.5709x; 1.7791x over previous
import jax
import jax.numpy as jnp
from jax import lax
from jax.experimental import pallas as pl
from jax.experimental.pallas import tpu as pltpu
from jax.experimental.pallas import tpu_sc as plsc

N = 10000
E = 320000
D_IN = 128
HEADS = 8
C1 = 8
D_OUT = 16
NEG_SLOPE = 0.2

NC = 2
NS = 16
NW = NC * NS
CH = 128
NP = 10240
ROWS_PER_TILE = NP // NS
CHUNKS = 82
EP = NW * CHUNKS * CH
PER_W = CHUNKS * CH
W1ACC = 80
W2ACC = 32


def _leaky(t):
    return jnp.where(t >= 0, t, t * NEG_SLOPE)


def _tc1_body(x_ref, w1t_ref, as_ref, ad_ref, psig_ref, h1t_ref, tabs_ref,
              tabd_ref, g_ref):
    x = x_ref[...]
    h1t = jnp.dot(x, w1t_ref[...], preferred_element_type=jnp.float32)
    tabs = jnp.dot(h1t, as_ref[...], preferred_element_type=jnp.float32)
    tabd = jnp.dot(h1t, ad_ref[...], preferred_element_type=jnp.float32)
    h1t_ref[...] = jnp.dot(h1t, psig_ref[...],
                           preferred_element_type=jnp.float32
                           ).astype(jnp.bfloat16)
    tabs_ref[...] = tabs
    tabd_ref[...] = tabd
    g = _leaky(jnp.max(tabs, axis=0, keepdims=True)
               + jnp.max(tabd, axis=0, keepdims=True))
    g_ref[...] = jnp.broadcast_to(g, (8, 16))


def _tc1(xp, W1T, AsT, AdT, Psig):
    return pl.pallas_call(
        _tc1_body,
        out_shape=[
            jax.ShapeDtypeStruct((NP, HEADS * C1), jnp.bfloat16),
            jax.ShapeDtypeStruct((NP, 16), jnp.float32),
            jax.ShapeDtypeStruct((NP, 16), jnp.float32),
            jax.ShapeDtypeStruct((8, 16), jnp.float32),
        ],
    )(xp, W1T, AsT, AdT, Psig)


def _sc1_body(src_hbm, dst_hbm, tabs_hbm, tabd_hbm, h1t_hbm, g_hbm, out_hbm,
              srcidx, dstidx, srows, drows, hrows, msgbuf, gbuf, acc,
              gsem0, gsem1, ssem0, ssem1):
    c = lax.axis_index("c")
    s = lax.axis_index("s")
    wid = s * NC + c
    zero16 = jnp.zeros((16,), jnp.float32)
    gsems = [gsem0, gsem1]
    ssems = [ssem0, ssem1]

    pltpu.sync_copy(g_hbm, gbuf)
    gvec = gbuf[...]
    pltpu.sync_copy(src_hbm.at[wid], srcidx)
    pltpu.sync_copy(dst_hbm.at[wid], dstidx)

    mb0 = msgbuf.at[0]

    def _zb(t, _):
        mb0[t // 5, pl.ds((t % 5) * 16, 16)] = zero16
        return 0
    lax.fori_loop(0, CH * 5, _zb, 0)
    for q in range(ROWS_PER_TILE // CH):
        pltpu.sync_copy(mb0, acc.at[pl.ds(s * ROWS_PER_TILE + q * CH, CH)])
    plsc.subcore_barrier()

    def _issue(t, b):
        pltpu.async_copy(tabs_hbm.at[srcidx.at[t]], srows.at[b], gsems[b])
        pltpu.async_copy(tabd_hbm.at[dstidx.at[t]], drows.at[b], gsems[b])
        pltpu.async_copy(h1t_hbm.at[srcidx.at[t]], hrows.at[b], gsems[b])

    def _wait(t, b):
        pltpu.make_async_copy(tabs_hbm.at[srcidx.at[t]], srows.at[b],
                              gsems[b]).wait()
        pltpu.make_async_copy(tabd_hbm.at[dstidx.at[t]], drows.at[b],
                              gsems[b]).wait()
        pltpu.make_async_copy(h1t_hbm.at[srcidx.at[t]], hrows.at[b],
                              gsems[b]).wait()

    _issue(0, 0)
    _issue(1, 1)

    def _step(t2, _):
        for b in range(2):
            t = 2 * t2 + b
            sr = srows.at[b]
            dr = drows.at[b]
            hr = hrows.at[b]
            mb = msgbuf.at[b]
            _wait(t, b)

            @pl.when(t >= 2)
            def _():
                pltpu.make_async_copy(mb, acc.at[dstidx.at[t]],
                                      ssems[b]).wait()

            @plsc.parallel_loop(0, CH, 1, unroll=8)
            def _edge(k):
                rs = sr[k, pl.ds(0, 16)]
                rd = dr[k, pl.ds(0, 16)]
                e = jnp.exp(_leaky(rs + rd) - gvec)
                mb[k, pl.ds(64, 16)] = e
                q0, q1 = plsc.unpack(hr[k, pl.ds(0, 32)],
                                     format=plsc.PackFormat.INTERLEAVED)
                q2, q3 = plsc.unpack(hr[k, pl.ds(32, 32)],
                                     format=plsc.PackFormat.INTERLEAVED)
                mb[k, pl.ds(0, 16)] = e * q0
                mb[k, pl.ds(16, 16)] = e * q1
                mb[k, pl.ds(32, 16)] = e * q2
                mb[k, pl.ds(48, 16)] = e * q3

            pltpu.async_copy(mb, acc.at[dstidx.at[t]], ssems[b], add=True)

            @pl.when(t + 2 < CHUNKS)
            def _():
                _issue(t + 2, b)
        return 0

    lax.fori_loop(0, CHUNKS // 2, _step, 0)
    for b in range(2):
        pltpu.make_async_copy(msgbuf.at[b], acc.at[dstidx.at[CHUNKS - 2 + b]],
                              ssems[b]).wait()
    plsc.subcore_barrier()
    pltpu.sync_copy(acc.at[pl.ds(s * ROWS_PER_TILE, ROWS_PER_TILE)],
                    out_hbm.at[c, pl.ds(s * ROWS_PER_TILE, ROWS_PER_TILE)])


def _sc1(src, dst, tabs, tabd, h1t, g16):
    mesh = plsc.VectorSubcoreMesh(core_axis_name="c", subcore_axis_name="s")
    f = pl.kernel(
        _sc1_body,
        out_type=jax.ShapeDtypeStruct((NC, NP, W1ACC), jnp.float32),
        mesh=mesh,
        compiler_params=pltpu.CompilerParams(needs_layout_passes=False,
                                             use_tc_tiling_on_sc=False),
        scratch_types=[
            pltpu.VMEM((CHUNKS, CH), jnp.int32),
            pltpu.VMEM((CHUNKS, CH), jnp.int32),
            pltpu.VMEM((2, CH, 16), jnp.float32),
            pltpu.VMEM((2, CH, 16), jnp.float32),
            pltpu.VMEM((2, CH, HEADS * C1), jnp.bfloat16),
            pltpu.VMEM((2, CH, W1ACC), jnp.float32),
            pltpu.VMEM((16,), jnp.float32),
            pltpu.VMEM_SHARED((NP, W1ACC), jnp.float32),
            pltpu.SemaphoreType.DMA,
            pltpu.SemaphoreType.DMA,
            pltpu.SemaphoreType.DMA,
            pltpu.SemaphoreType.DMA,
        ],
    )
    return f(src, dst, tabs, tabd, h1t, g16)


def _tc2_body(p_ref, b1_ref, gam_ref, bet_ref, w2_ref, a2w_ref, e8t_ref,
              perm_ref, h2_ref, a2_ref, g_ref):
    m = p_ref[0] + p_ref[1]
    num = m[:, :64]
    den = m[:, 64:72]
    r = 1.0 / (den + 1e-16)
    hcm = num * jnp.dot(r, e8t_ref[...], preferred_element_type=jnp.float32)
    h = jnp.dot(hcm, perm_ref[...], preferred_element_type=jnp.float32)
    h = h + b1_ref[...][None, :]
    h = jnp.maximum(h, 0.0)
    rows = lax.broadcasted_iota(jnp.int32, (NP, 1), 0)
    mask = rows < N
    hm = jnp.where(mask, h, 0.0)
    mean = jnp.sum(hm, axis=0, keepdims=True) / N
    d = jnp.where(mask, h - mean, 0.0)
    var = jnp.sum(d * d, axis=0, keepdims=True) / N
    hbn = (h - mean) * lax.rsqrt(var + 1e-5) * gam_ref[...][None, :] \
        + bet_ref[...][None, :]
    hbn = jnp.where(mask, hbn, 0.0)
    h2 = jnp.dot(hbn, w2_ref[...], preferred_element_type=jnp.float32)
    a2 = jnp.dot(h2, a2w_ref[...], preferred_element_type=jnp.float32)
    h2_ref[...] = h2
    a2_ref[...] = a2
    g = _leaky(jnp.max(a2[:, 0:1], axis=0, keepdims=True)
               + jnp.max(a2[:, 1:2], axis=0, keepdims=True))
    g_ref[...] = jnp.broadcast_to(g, (8, 16))


def _tc2(part1, b1, bn_gamma, bn_beta, W2, A2, E8T, P):
    return pl.pallas_call(
        _tc2_body,
        out_shape=[
            jax.ShapeDtypeStruct((NP, D_OUT), jnp.float32),
            jax.ShapeDtypeStruct((NP, 8), jnp.float32),
            jax.ShapeDtypeStruct((8, 16), jnp.float32),
        ],
    )(part1, b1, bn_gamma, bn_beta, W2, A2, E8T, P)


def _sc2_body(src_hbm, dst_hbm, asrc_hbm, adst_hbm, h2_hbm, g_hbm, out_hbm,
              srcidx, dstidx, asrcbuf, adstbuf, hrows, exbuf, msgbuf, gbuf,
              acc, gsem0, gsem1, ssem0, ssem1):
    c = lax.axis_index("c")
    s = lax.axis_index("s")
    wid = s * NC + c
    zero16 = jnp.zeros((16,), jnp.float32)
    gsems = [gsem0, gsem1]
    ssems = [ssem0, ssem1]

    pltpu.sync_copy(g_hbm, gbuf)
    pltpu.sync_copy(asrc_hbm, asrcbuf)
    pltpu.sync_copy(adst_hbm, adstbuf)
    g0 = gbuf[...][0]
    pltpu.sync_copy(src_hbm.at[wid], srcidx)
    pltpu.sync_copy(dst_hbm.at[wid], dstidx)

    mb0 = msgbuf.at[0]

    def _zb(t, _):
        mb0[t >> 1, pl.ds((t & 1) * 16, 16)] = zero16
        return 0
    lax.fori_loop(0, CH * 2, _zb, 0)
    for q in range(ROWS_PER_TILE // CH):
        pltpu.sync_copy(mb0, acc.at[pl.ds(s * ROWS_PER_TILE + q * CH, CH)])
    plsc.subcore_barrier()

    def _issue(t, b):
        pltpu.async_copy(h2_hbm.at[srcidx.at[t]], hrows.at[b], gsems[b])

    def _wait(t, b):
        pltpu.make_async_copy(h2_hbm.at[srcidx.at[t]], hrows.at[b],
                              gsems[b]).wait()

    _issue(0, 0)
    _issue(1, 1)

    def _step(t2, _):
        for b in range(2):
            t = 2 * t2 + b
            hr = hrows.at[b]
            mb = msgbuf.at[b]

            @plsc.parallel_loop(0, CH // 16, 1, unroll=2)
            def _exb(i):
                sv = srcidx[t, pl.ds(i * 16, 16)]
                dv = dstidx[t, pl.ds(i * 16, 16)]
                u = plsc.load_gather(asrcbuf, [sv])
                v = plsc.load_gather(adstbuf, [dv])
                e = jnp.exp(_leaky(u + v) - g0)
                exbuf[pl.ds(b * CH + i * 16, 16)] = e
            _wait(t, b)

            @pl.when(t >= 2)
            def _():
                pltpu.make_async_copy(mb, acc.at[dstidx.at[t]],
                                      ssems[b]).wait()

            @plsc.parallel_loop(0, CH, 1, unroll=4)
            def _msg(k):
                eb = plsc.load_gather(
                    exbuf, [jnp.full((16,), b * CH + k, jnp.int32)])
                hv = hr[k, pl.ds(0, 16)]
                mb[k, pl.ds(0, 16)] = eb * hv
                mb[k, pl.ds(16, 16)] = eb

            pltpu.async_copy(mb, acc.at[dstidx.at[t]], ssems[b], add=True)

            @pl.when(t + 2 < CHUNKS)
            def _():
                _issue(t + 2, b)
        return 0

    lax.fori_loop(0, CHUNKS // 2, _step, 0)
    for b in range(2):
        pltpu.make_async_copy(msgbuf.at[b], acc.at[dstidx.at[CHUNKS - 2 + b]],
                              ssems[b]).wait()
    plsc.subcore_barrier()
    pltpu.sync_copy(acc.at[pl.ds(s * ROWS_PER_TILE, ROWS_PER_TILE)],
                    out_hbm.at[c, pl.ds(s * ROWS_PER_TILE, ROWS_PER_TILE)])


def _sc2(src, dst, asrc2, adst2, h2, g16):
    mesh = plsc.VectorSubcoreMesh(core_axis_name="c", subcore_axis_name="s")
    f = pl.kernel(
        _sc2_body,
        out_type=jax.ShapeDtypeStruct((NC, NP, W2ACC), jnp.float32),
        mesh=mesh,
        compiler_params=pltpu.CompilerParams(needs_layout_passes=False,
                                             use_tc_tiling_on_sc=False),
        scratch_types=[
            pltpu.VMEM((CHUNKS, CH), jnp.int32),
            pltpu.VMEM((CHUNKS, CH), jnp.int32),
            pltpu.VMEM((NP,), jnp.float32),
            pltpu.VMEM((NP,), jnp.float32),
            pltpu.VMEM((2, CH, D_OUT), jnp.float32),
            pltpu.VMEM((2 * CH,), jnp.float32),
            pltpu.VMEM((2, CH, W2ACC), jnp.float32),
            pltpu.VMEM((16,), jnp.float32),
            pltpu.VMEM_SHARED((NP, W2ACC), jnp.float32),
            pltpu.SemaphoreType.DMA,
            pltpu.SemaphoreType.DMA,
            pltpu.SemaphoreType.DMA,
            pltpu.SemaphoreType.DMA,
        ],
    )
    return f(src, dst, asrc2, adst2, h2, g16)


def _tc3_body(p_ref, b2_ref, out_ref):
    m = p_ref[0] + p_ref[1]
    num = m[:, :16]
    den = m[:, 16:17]
    out_ref[...] = num / (den + 1e-16) + b2_ref[...][None, :]


def _tc3(part2, b2):
    return pl.pallas_call(
        _tc3_body,
        out_shape=jax.ShapeDtypeStruct((NP, D_OUT), jnp.float32),
    )(part2, b2)


def kernel(x, edge_index, W1, att_src1, att_dst1, b1, bn_gamma, bn_beta,
           W2, att_src2, att_dst2, b2):
    xp = jnp.pad(x, ((0, NP - N), (0, 0)))
    ei = edge_index.astype(jnp.int32)
    loops = jnp.arange(N, dtype=jnp.int32)
    padi = jnp.full((EP - E - N,), N, jnp.int32)
    src = jnp.concatenate([ei[0], loops, padi]).reshape(NW, CHUNKS, CH)
    dst = jnp.concatenate([ei[1], loops, padi]).reshape(NW, CHUNKS, CH)

    eye8 = jnp.eye(8, dtype=jnp.float32)
    P = (eye8[:, None, None, :] * eye8[None, :, :, None]).reshape(64, 64)
    W1T = jnp.dot(W1, P)
    AsT = (att_src1.T[:, :, None] * eye8[None, :, :]).reshape(64, 8)
    AdT = (att_dst1.T[:, :, None] * eye8[None, :, :]).reshape(64, 8)
    AsTD = jnp.concatenate([AsT, AsT], axis=1)
    AdTD = jnp.concatenate([AdT, AdT], axis=1)
    E8T = jnp.tile(eye8, (1, 8))
    A2 = jnp.zeros((D_OUT, 8), jnp.float32)
    A2 = A2.at[:, 0].set(att_src2[0]).at[:, 1].set(att_dst2[0])
    cols = jnp.arange(64)
    bfcol = jnp.where(cols < 16, 2 * cols,
             jnp.where(cols < 32, 2 * (cols - 16) + 1,
              jnp.where(cols < 48, 32 + 2 * (cols - 32),
                        32 + 2 * (cols - 48) + 1)))
    Psig = jnp.zeros((64, 64), jnp.float32).at[cols, bfcol].set(1.0)

    h1t, tabs, tabd, g88 = _tc1(xp, W1T, AsTD, AdTD, Psig)
    g16 = g88[0]
    part1 = _sc1(src, dst, tabs, tabd, h1t, g16)
    h2, a2, g2_88 = _tc2(part1, b1, bn_gamma, bn_beta, W2, A2, E8T, P)
    g2_16 = g2_88[0]
    part2 = _sc2(src, dst, a2[:, 0], a2[:, 1], h2, g2_16)
    out = _tc3(part2, b2)
    return out[:N]
